# baseline (speedup 1.0000x reference)
import jax
import jax.numpy as jnp
from jax.experimental import pallas as pl
from jax.experimental.pallas import tpu as pltpu

DIM = 192
TIME = 32
HW = 1024
D6 = DIM // 6
DSH = 2 * D6


def _rope_body(tidx_ref, ttab_c_ref, ttab_s_ref, spat_c_ref, spat_s_ref,
               cos_ref, sin_ref):
    b = pl.program_id(0)
    s = pl.program_id(1)
    idx = tidx_ref[b, s]
    trow_c = ttab_c_ref[pl.ds(idx, 1), :]
    trow_s = ttab_s_ref[pl.ds(idx, 1), :]
    cos_ref[0] = spat_c_ref[...] + trow_c
    sin_ref[0] = spat_s_ref[...] + trow_s


def kernel(t_idxs, cos_t, sin_t, cos_s, sin_s):
    B, S = t_idxs.shape
    zt = jnp.zeros((TIME, DSH), jnp.float32)
    ttab_c = jnp.concatenate([cos_t, zt, cos_t, zt], axis=1)
    ttab_s = jnp.concatenate([sin_t, zt, sin_t, zt], axis=1)
    zs = jnp.zeros((HW, D6), jnp.float32)
    spat_c = jnp.concatenate([zs, cos_s, zs, cos_s], axis=1)
    spat_s = jnp.concatenate([zs, sin_s, zs, sin_s], axis=1)

    grid_spec = pltpu.PrefetchScalarGridSpec(
        num_scalar_prefetch=1,
        grid=(B, S),
        in_specs=[
            pl.BlockSpec((TIME, DIM), lambda b, s, tidx: (0, 0)),
            pl.BlockSpec((TIME, DIM), lambda b, s, tidx: (0, 0)),
            pl.BlockSpec((HW, DIM), lambda b, s, tidx: (0, 0)),
            pl.BlockSpec((HW, DIM), lambda b, s, tidx: (0, 0)),
        ],
        out_specs=[
            pl.BlockSpec((1, HW, DIM), lambda b, s, tidx: (b, s, 0)),
            pl.BlockSpec((1, HW, DIM), lambda b, s, tidx: (b, s, 0)),
        ],
    )
    out_shape = jax.ShapeDtypeStruct((B, S * HW, DIM), jnp.float32)
    cos, sin = pl.pallas_call(
        _rope_body,
        grid_spec=grid_spec,
        out_shape=[out_shape, out_shape],
        compiler_params=pltpu.CompilerParams(
            dimension_semantics=("parallel", "parallel")),
    )(t_idxs.astype(jnp.int32), ttab_c, ttab_s, spat_c, spat_s)
    return (cos, sin)


# speedup vs baseline: 1.0252x; 1.0252x over previous
import jax
import jax.numpy as jnp
from jax.experimental import pallas as pl
from jax.experimental.pallas import tpu as pltpu

DIM = 192
TIME = 32
HW = 1024
D6 = DIM // 6
DSH = 2 * D6


S_BLK = 4


def _rope_body(tidx_ref, ttab_c_ref, ttab_s_ref, spat_c_ref, spat_s_ref,
               cos_ref, sin_ref):
    b = pl.program_id(0)
    j = pl.program_id(1)
    spat_c = spat_c_ref[...]
    spat_s = spat_s_ref[...]
    for u in range(S_BLK):
        idx = tidx_ref[b, j * S_BLK + u]
        trow_c = ttab_c_ref[pl.ds(idx, 1), :]
        trow_s = ttab_s_ref[pl.ds(idx, 1), :]
        cos_ref[0, pl.ds(u * HW, HW), :] = spat_c + trow_c
        sin_ref[0, pl.ds(u * HW, HW), :] = spat_s + trow_s


def kernel(t_idxs, cos_t, sin_t, cos_s, sin_s):
    B, S = t_idxs.shape
    zt = jnp.zeros((TIME, DSH), jnp.float32)
    ttab_c = jnp.concatenate([cos_t, zt, cos_t, zt], axis=1)
    ttab_s = jnp.concatenate([sin_t, zt, sin_t, zt], axis=1)
    zs = jnp.zeros((HW, D6), jnp.float32)
    spat_c = jnp.concatenate([zs, cos_s, zs, cos_s], axis=1)
    spat_s = jnp.concatenate([zs, sin_s, zs, sin_s], axis=1)

    grid_spec = pltpu.PrefetchScalarGridSpec(
        num_scalar_prefetch=1,
        grid=(B, S // S_BLK),
        in_specs=[
            pl.BlockSpec((TIME, DIM), lambda b, s, tidx: (0, 0)),
            pl.BlockSpec((TIME, DIM), lambda b, s, tidx: (0, 0)),
            pl.BlockSpec((HW, DIM), lambda b, s, tidx: (0, 0)),
            pl.BlockSpec((HW, DIM), lambda b, s, tidx: (0, 0)),
        ],
        out_specs=[
            pl.BlockSpec((1, S_BLK * HW, DIM), lambda b, s, tidx: (b, s, 0)),
            pl.BlockSpec((1, S_BLK * HW, DIM), lambda b, s, tidx: (b, s, 0)),
        ],
    )
    out_shape = jax.ShapeDtypeStruct((B, S * HW, DIM), jnp.float32)
    cos, sin = pl.pallas_call(
        _rope_body,
        grid_spec=grid_spec,
        out_shape=[out_shape, out_shape],
        compiler_params=pltpu.CompilerParams(
            dimension_semantics=("parallel", "parallel")),
    )(t_idxs.astype(jnp.int32), ttab_c, ttab_s, spat_c, spat_s)
    return (cos, sin)
